# baseline (speedup 1.0000x reference)
import jax
import jax.numpy as jnp
from jax import lax
from jax.experimental import pallas as pl
from jax.experimental.pallas import tpu as pltpu
from jax.experimental.pallas import tpu_sc as plsc

_E = 800000
_T = 8
_C = 8
_L = 16
_NW = 32
_B = 640
_K = 4
_Z = _B // _K
_NSUP = _E // _B
_MAXIT = -(-_NSUP // _NW)
_G = _Z // _L


def _sc_body(it_h, oc_h, tt_h, ct_h, lut_h, td_h, cd_h, out_h,
             it_v, oc_v, tt_v, ct_v, lut_v, td_v, cd_v, out_v,
             in_sem, fat_sem, out_sem):
    wid = lax.axis_index("s") * 2 + lax.axis_index("c")
    lane = jnp.arange(_L, dtype=jnp.int32)
    eps = jnp.float32(1e-12)

    def fire_small(sc, b):
        base = sc * _B
        d = pl.ds(b * _B, _B)
        pltpu.async_copy(it_h.at[pl.ds(base, _B)], it_v.at[d], in_sem.at[b])
        pltpu.async_copy(oc_h.at[pl.ds(base, _B)], oc_v.at[d], in_sem.at[b])
        pltpu.async_copy(td_h.at[pl.ds(base, _B)], td_v.at[d], in_sem.at[b])
        pltpu.async_copy(cd_h.at[pl.ds(base, _B)], cd_v.at[d], in_sem.at[b])

    def wait_small(sc, b):
        base = sc * _B
        d = pl.ds(b * _B, _B)
        pltpu.make_async_copy(it_h.at[pl.ds(base, _B)], it_v.at[d], in_sem.at[b]).wait()
        pltpu.make_async_copy(oc_h.at[pl.ds(base, _B)], oc_v.at[d], in_sem.at[b]).wait()
        pltpu.make_async_copy(td_h.at[pl.ds(base, _B)], td_v.at[d], in_sem.at[b]).wait()
        pltpu.make_async_copy(cd_h.at[pl.ds(base, _B)], cd_v.at[d], in_sem.at[b]).wait()

    def fire_fat(abase, bl):
        d = pl.ds(bl * _Z, _Z)
        pltpu.async_copy(tt_h.at[pl.ds(abase, _Z)], tt_v.at[d], fat_sem.at[bl])
        pltpu.async_copy(ct_h.at[pl.ds(abase, _Z)], ct_v.at[d], fat_sem.at[bl])
        pltpu.async_copy(lut_h.at[pl.ds(abase, _Z)], lut_v.at[d], fat_sem.at[bl])

    def wait_fat(abase, bl):
        d = pl.ds(bl * _Z, _Z)
        pltpu.make_async_copy(tt_h.at[pl.ds(abase, _Z)], tt_v.at[d], fat_sem.at[bl]).wait()
        pltpu.make_async_copy(ct_h.at[pl.ds(abase, _Z)], ct_v.at[d], fat_sem.at[bl]).wait()
        pltpu.make_async_copy(lut_h.at[pl.ds(abase, _Z)], lut_v.at[d], fat_sem.at[bl]).wait()

    def wait_out(sc, b):
        pltpu.make_async_copy(
            out_v.at[pl.ds(b * _B, _B)], out_h.at[pl.ds(sc * _B, _B)],
            out_sem.at[b]).wait()

    def compute_sub(bs, k, bl):
        soff = bs * _B + k * _Z
        foff = bl * _Z

        @plsc.parallel_loop(0, _G, 1, unroll=2)
        def g_body(g):
            s = soff + g * _L
            frows = foff + g * _L + lane
            it = it_v[pl.ds(s, _L)]
            oc = oc_v[pl.ds(s, _L)]
            td = td_v[pl.ds(s, _L)]
            cd = cd_v[pl.ds(s, _L)]

            t_idx = jnp.zeros((_L,), jnp.int32)
            c_idx = jnp.zeros((_L,), jnp.int32)
            for j in range(_T):
                col = jnp.full((_L,), j, jnp.int32)
                ttj = plsc.load_gather(tt_v, [frows, col])
                ctj = plsc.load_gather(ct_v, [frows, col])
                t_idx = t_idx + (ttj <= it).astype(jnp.int32)
                c_idx = c_idx + (ctj <= oc).astype(jnp.int32)

            max_t = jnp.maximum(td - 1, 0)
            max_c = jnp.maximum(cd - 1, 0)
            t_hi = jnp.minimum(jnp.maximum(t_idx, 1), max_t)
            c_hi = jnp.minimum(jnp.maximum(c_idx, 1), max_c)
            t_lo = t_hi - 1
            c_lo = c_hi - 1

            t0 = plsc.load_gather(tt_v, [frows, t_lo])
            t1 = plsc.load_gather(tt_v, [frows, t_hi])
            c0 = plsc.load_gather(ct_v, [frows, c_lo])
            c1 = plsc.load_gather(ct_v, [frows, c_hi])

            o00 = t_lo * cd + c_lo
            o10 = o00 + cd
            v00 = plsc.load_gather(lut_v, [frows, o00])
            v01 = plsc.load_gather(lut_v, [frows, o00 + 1])
            v10 = plsc.load_gather(lut_v, [frows, o10])
            v11 = plsc.load_gather(lut_v, [frows, o10 + 1])

            t_int = t1 - t0
            c_int = c1 - c0
            t_deg = jnp.abs(t_int) < eps
            c_deg = jnp.abs(c_int) < eps
            x = jnp.clip(it, t0, t1)
            y = jnp.clip(oc, c0, c1)
            ts = jnp.where(t_deg, eps, t_int)
            cs = jnp.where(c_deg, eps, c_int)
            rt = jnp.float32(1.0) / ts
            rc = jnp.float32(1.0) / cs
            rd = rt * rc
            dx0 = x - t0
            dx1 = t1 - x
            dy0 = y - c0
            dy1 = c1 - y
            b00 = dx1 * dy1 * rd
            b01 = dx1 * dy0 * rd
            b10 = dx0 * dy1 * rd
            b11 = dx0 * dy0 * rd
            fc = jnp.clip(dy0 * rc, 0.0, 1.0)
            ft = jnp.clip(dx0 * rt, 0.0, 1.0)
            one = jnp.float32(1.0)
            zero = jnp.float32(0.0)
            a00 = jnp.where(t_deg, jnp.where(c_deg, one, one - fc),
                            jnp.where(c_deg, one - ft, b00))
            a01 = jnp.where(t_deg, jnp.where(c_deg, zero, fc),
                            jnp.where(c_deg, zero, b01))
            a10 = jnp.where(t_deg, zero, jnp.where(c_deg, ft, b10))
            a11 = jnp.where(t_deg, zero, jnp.where(c_deg, zero, b11))

            out_v[pl.ds(s, _L)] = (a00 * v00 + a01 * v01
                                   + a10 * v10 + a11 * v11)

    fire_small(wid, 0)
    fire_fat(wid * _B, 0)

    def sup_body(i, carry):
        sc = wid + i * _NW
        bs = jnp.bitwise_and(i, 1)

        @pl.when(sc < _NSUP)
        def _():
            sc_next = sc + _NW

            @pl.when(sc_next < _NSUP)
            def _():
                fire_small(sc_next, 1 - bs)

            wait_small(sc, bs)

            @pl.when(i >= 2)
            def _():
                wait_out(sc - 2 * _NW, bs)

            for k in range(_K):
                bl = k & 1
                if k + 1 < _K:
                    fire_fat(sc * _B + (k + 1) * _Z, 1 - bl)
                else:
                    @pl.when(sc_next < _NSUP)
                    def _():
                        fire_fat(sc_next * _B, 1 - bl)

                wait_fat(sc * _B + k * _Z, bl)
                compute_sub(bs, k, bl)

            pltpu.async_copy(out_v.at[pl.ds(bs * _B, _B)],
                             out_h.at[pl.ds(sc * _B, _B)], out_sem.at[bs])

        return carry

    lax.fori_loop(0, _MAXIT, sup_body, 0)

    n_i = lax.shift_right_logical(_NSUP - wid + _NW - 1, 5)

    def drain(k, carry):
        i = n_i - 2 + k

        @pl.when(i >= 0)
        def _():
            wait_out(wid + i * _NW, jnp.bitwise_and(i, 1))

        return carry

    lax.fori_loop(0, 2, drain, 0)


@jax.jit
def _sc_call(input_trans, output_caps, trans_tables, cap_tables, lut_values,
             trans_dims, cap_dims):
    mesh = plsc.VectorSubcoreMesh(core_axis_name="c", subcore_axis_name="s")
    f = pl.kernel(
        _sc_body,
        out_type=jax.ShapeDtypeStruct((_E,), jnp.float32),
        mesh=mesh,
        compiler_params=pltpu.CompilerParams(
            needs_layout_passes=False, disable_bounds_checks=True),
        scratch_types=[
            pltpu.VMEM((2 * _B,), jnp.float32),
            pltpu.VMEM((2 * _B,), jnp.float32),
            pltpu.VMEM((2 * _Z, _T), jnp.float32),
            pltpu.VMEM((2 * _Z, _C), jnp.float32),
            pltpu.VMEM((2 * _Z, _T * _C), jnp.float32),
            pltpu.VMEM((2 * _B,), jnp.int32),
            pltpu.VMEM((2 * _B,), jnp.int32),
            pltpu.VMEM((2 * _B,), jnp.float32),
            pltpu.SemaphoreType.DMA((2,)),
            pltpu.SemaphoreType.DMA((2,)),
            pltpu.SemaphoreType.DMA((2,)),
        ],
    )
    return f(input_trans, output_caps, trans_tables, cap_tables, lut_values,
             trans_dims, cap_dims)


def kernel(input_trans, output_caps, trans_tables, cap_tables, lut_values,
           trans_dims, cap_dims):
    return _sc_call(input_trans, output_caps, trans_tables, cap_tables,
                    lut_values, trans_dims, cap_dims)


# speedup vs baseline: 1.1197x; 1.1197x over previous
import jax
import jax.numpy as jnp
from jax import lax
from jax.experimental import pallas as pl
from jax.experimental.pallas import tpu as pltpu
from jax.experimental.pallas import tpu_sc as plsc

_E = 800000
_T = 8
_C = 8
_L = 16
_NW = 32
_B = 320
_K = 4
_Z = _B // _K
_NSUP = _E // _B
_MAXIT = -(-_NSUP // _NW)
_G = _Z // _L


def _sc_body(it_h, oc_h, tt_h, ct_h, lut_h, td_h, cd_h, out_h,
             it_v, oc_v, tt_v, ct_v, lut_v, td_v, cd_v, out_v,
             in_sem, tab_sem, fat_sem, out_sem):
    wid = lax.axis_index("s") * 2 + lax.axis_index("c")
    lane = jnp.arange(_L, dtype=jnp.int32)
    eps = jnp.float32(1e-12)

    def fire_small(sc, b):
        base = sc * _B
        d = pl.ds(b * _B, _B)
        pltpu.async_copy(it_h.at[pl.ds(base, _B)], it_v.at[d], in_sem.at[b])
        pltpu.async_copy(oc_h.at[pl.ds(base, _B)], oc_v.at[d], in_sem.at[b])
        pltpu.async_copy(td_h.at[pl.ds(base, _B)], td_v.at[d], in_sem.at[b])
        pltpu.async_copy(cd_h.at[pl.ds(base, _B)], cd_v.at[d], in_sem.at[b])

    def wait_small(sc, b):
        base = sc * _B
        d = pl.ds(b * _B, _B)
        pltpu.make_async_copy(it_h.at[pl.ds(base, _B)], it_v.at[d], in_sem.at[b]).wait()
        pltpu.make_async_copy(oc_h.at[pl.ds(base, _B)], oc_v.at[d], in_sem.at[b]).wait()
        pltpu.make_async_copy(td_h.at[pl.ds(base, _B)], td_v.at[d], in_sem.at[b]).wait()
        pltpu.make_async_copy(cd_h.at[pl.ds(base, _B)], cd_v.at[d], in_sem.at[b]).wait()

    def fire_tab(abase, bt):
        d = pl.ds(bt * _Z, _Z)
        pltpu.async_copy(tt_h.at[pl.ds(abase, _Z)], tt_v.at[d], tab_sem.at[bt])
        pltpu.async_copy(ct_h.at[pl.ds(abase, _Z)], ct_v.at[d], tab_sem.at[bt])

    def wait_tab(abase, bt):
        d = pl.ds(bt * _Z, _Z)
        pltpu.make_async_copy(tt_h.at[pl.ds(abase, _Z)], tt_v.at[d], tab_sem.at[bt]).wait()
        pltpu.make_async_copy(ct_h.at[pl.ds(abase, _Z)], ct_v.at[d], tab_sem.at[bt]).wait()

    def fire_lut(abase, bl):
        pltpu.async_copy(lut_h.at[pl.ds(abase, _Z)],
                         lut_v.at[pl.ds(bl * _Z, _Z)], fat_sem.at[bl])

    def wait_lut(abase, bl):
        pltpu.make_async_copy(lut_h.at[pl.ds(abase, _Z)],
                              lut_v.at[pl.ds(bl * _Z, _Z)],
                              fat_sem.at[bl]).wait()

    def wait_out(sc, b):
        pltpu.make_async_copy(
            out_v.at[pl.ds(b * _B, _B)], out_h.at[pl.ds(sc * _B, _B)],
            out_sem.at[b]).wait()

    def compute_sub(bs, k, bt, bl):
        soff = bs * _B + k * _Z
        toff = bt * _Z
        foff = bl * _Z

        @plsc.parallel_loop(0, _G, 1, unroll=2)
        def g_body(g):
            s = soff + g * _L
            trows = toff + g * _L + lane
            frows = foff + g * _L + lane
            it = it_v[pl.ds(s, _L)]
            oc = oc_v[pl.ds(s, _L)]
            td = td_v[pl.ds(s, _L)]
            cd = cd_v[pl.ds(s, _L)]

            t_idx = jnp.zeros((_L,), jnp.int32)
            c_idx = jnp.zeros((_L,), jnp.int32)
            for j in range(_T):
                col = jnp.full((_L,), j, jnp.int32)
                ttj = plsc.load_gather(tt_v, [trows, col])
                ctj = plsc.load_gather(ct_v, [trows, col])
                t_idx = t_idx + (ttj <= it).astype(jnp.int32)
                c_idx = c_idx + (ctj <= oc).astype(jnp.int32)

            max_t = jnp.maximum(td - 1, 0)
            max_c = jnp.maximum(cd - 1, 0)
            t_hi = jnp.minimum(jnp.maximum(t_idx, 1), max_t)
            c_hi = jnp.minimum(jnp.maximum(c_idx, 1), max_c)
            t_lo = t_hi - 1
            c_lo = c_hi - 1

            t0 = plsc.load_gather(tt_v, [trows, t_lo])
            t1 = plsc.load_gather(tt_v, [trows, t_hi])
            c0 = plsc.load_gather(ct_v, [trows, c_lo])
            c1 = plsc.load_gather(ct_v, [trows, c_hi])

            o00 = t_lo * cd + c_lo
            o10 = o00 + cd
            v00 = plsc.load_gather(lut_v, [frows, o00])
            v01 = plsc.load_gather(lut_v, [frows, o00 + 1])
            v10 = plsc.load_gather(lut_v, [frows, o10])
            v11 = plsc.load_gather(lut_v, [frows, o10 + 1])

            t_int = t1 - t0
            c_int = c1 - c0
            t_deg = jnp.abs(t_int) < eps
            c_deg = jnp.abs(c_int) < eps
            x = jnp.clip(it, t0, t1)
            y = jnp.clip(oc, c0, c1)
            ts = jnp.where(t_deg, eps, t_int)
            cs = jnp.where(c_deg, eps, c_int)
            rt = jnp.float32(1.0) / ts
            rc = jnp.float32(1.0) / cs
            rd = rt * rc
            dx0 = x - t0
            dx1 = t1 - x
            dy0 = y - c0
            dy1 = c1 - y
            b00 = dx1 * dy1 * rd
            b01 = dx1 * dy0 * rd
            b10 = dx0 * dy1 * rd
            b11 = dx0 * dy0 * rd
            fc = jnp.clip(dy0 * rc, 0.0, 1.0)
            ft = jnp.clip(dx0 * rt, 0.0, 1.0)
            one = jnp.float32(1.0)
            zero = jnp.float32(0.0)
            a00 = jnp.where(t_deg, jnp.where(c_deg, one, one - fc),
                            jnp.where(c_deg, one - ft, b00))
            a01 = jnp.where(t_deg, jnp.where(c_deg, zero, fc),
                            jnp.where(c_deg, zero, b01))
            a10 = jnp.where(t_deg, zero, jnp.where(c_deg, ft, b10))
            a11 = jnp.where(t_deg, zero, jnp.where(c_deg, zero, b11))

            out_v[pl.ds(s, _L)] = (a00 * v00 + a01 * v01
                                   + a10 * v10 + a11 * v11)

    fire_small(wid, 0)
    for _kk in range(3):
        fire_tab(wid * _B + _kk * _Z, _kk)
        fire_lut(wid * _B + _kk * _Z, _kk)

    def sup_body(i, carry):
        sc = wid + i * _NW
        bs = jnp.bitwise_and(i, 1)

        @pl.when(sc < _NSUP)
        def _():
            sc_next = sc + _NW

            @pl.when(sc_next < _NSUP)
            def _():
                fire_small(sc_next, 1 - bs)

            wait_small(sc, bs)

            @pl.when(i >= 2)
            def _():
                wait_out(sc - 2 * _NW, bs)

            for k in range(_K):
                dj3, kp3 = divmod(k + 3, _K)
                if dj3 == 0:
                    fire_tab(sc * _B + kp3 * _Z, kp3)
                    fire_lut(sc * _B + kp3 * _Z, kp3)
                else:
                    @pl.when(sc_next < _NSUP)
                    def _():
                        fire_tab(sc_next * _B + kp3 * _Z, kp3)
                        fire_lut(sc_next * _B + kp3 * _Z, kp3)

                wait_tab(sc * _B + k * _Z, k)
                wait_lut(sc * _B + k * _Z, k)
                compute_sub(bs, k, k, k)

            pltpu.async_copy(out_v.at[pl.ds(bs * _B, _B)],
                             out_h.at[pl.ds(sc * _B, _B)], out_sem.at[bs])

        return carry

    lax.fori_loop(0, _MAXIT, sup_body, 0)

    n_i = lax.shift_right_logical(_NSUP - wid + _NW - 1, 5)

    def drain(k, carry):
        i = n_i - 2 + k

        @pl.when(i >= 0)
        def _():
            wait_out(wid + i * _NW, jnp.bitwise_and(i, 1))

        return carry

    lax.fori_loop(0, 2, drain, 0)


@jax.jit
def _sc_call(input_trans, output_caps, trans_tables, cap_tables, lut_values,
             trans_dims, cap_dims):
    mesh = plsc.VectorSubcoreMesh(core_axis_name="c", subcore_axis_name="s")
    f = pl.kernel(
        _sc_body,
        out_type=jax.ShapeDtypeStruct((_E,), jnp.float32),
        mesh=mesh,
        compiler_params=pltpu.CompilerParams(
            needs_layout_passes=False, disable_bounds_checks=True),
        scratch_types=[
            pltpu.VMEM((2 * _B,), jnp.float32),
            pltpu.VMEM((2 * _B,), jnp.float32),
            pltpu.VMEM((4 * _Z, _T), jnp.float32),
            pltpu.VMEM((4 * _Z, _C), jnp.float32),
            pltpu.VMEM((4 * _Z, _T * _C), jnp.float32),
            pltpu.VMEM((2 * _B,), jnp.int32),
            pltpu.VMEM((2 * _B,), jnp.int32),
            pltpu.VMEM((2 * _B,), jnp.float32),
            pltpu.SemaphoreType.DMA((2,)),
            pltpu.SemaphoreType.DMA((4,)),
            pltpu.SemaphoreType.DMA((4,)),
            pltpu.SemaphoreType.DMA((2,)),
        ],
    )
    return f(input_trans, output_caps, trans_tables, cap_tables, lut_values,
             trans_dims, cap_dims)


def kernel(input_trans, output_caps, trans_tables, cap_tables, lut_values,
           trans_dims, cap_dims):
    return _sc_call(input_trans, output_caps, trans_tables, cap_tables,
                    lut_values, trans_dims, cap_dims)
